# baseline (speedup 1.0000x reference)
import jax
import jax.numpy as jnp
from jax import lax
from jax.experimental import pallas as pl
from jax.experimental.pallas import tpu as pltpu
from jax.experimental.pallas import tpu_sc as plsc

N = 10000
N_PAD = 10240
E = 320000
D = 64
NC = 2
NS = 16
LANES = 16
NW = NC * NS
K = 125
CHUNKS = E // (NW * K)
NB = 5
ROUNDS = CHUNKS // NB
RPT = N_PAD // NS
ZR = 64
DEGW = 16


def _sc_segsum(num_tables, with_deg):
  mesh = plsc.VectorSubcoreMesh(
      core_axis_name="c", subcore_axis_name="s",
      num_cores=NC, num_subcores=NS)
  out_types = [jax.ShapeDtypeStruct((NC, N_PAD, D), jnp.float32)
               for _ in range(num_tables)]
  scratch = [
      pltpu.VMEM((CHUNKS, K), jnp.int32),
      pltpu.VMEM((CHUNKS, K), jnp.int32),
      [pltpu.VMEM((K, D), jnp.float32) for _ in range(NB)],
      [pltpu.SemaphoreType.DMA for _ in range(NB)],
      [pltpu.SemaphoreType.DMA for _ in range(NB)],
      pltpu.VMEM((ZR, D), jnp.float32),
      pltpu.VMEM((ZR, D), jnp.float32),
      pltpu.VMEM_SHARED((N_PAD, D), jnp.float32),
  ]
  if with_deg:
    out_types.append(jax.ShapeDtypeStruct((NC, N_PAD, DEGW), jnp.float32))
    scratch += [
        pltpu.VMEM((K, DEGW), jnp.float32),
        pltpu.VMEM((RPT // 4, DEGW), jnp.float32),
        pltpu.VMEM_SHARED((N_PAD, DEGW), jnp.float32),
        pltpu.SemaphoreType.DMA,
    ]

  def body(*refs):
    tables = refs[:num_tables]
    srcs, dsts = refs[num_tables:num_tables + 2]
    rest = refs[num_tables + 2:]
    outs = rest[:num_tables]
    rest = rest[num_tables:]
    if with_deg:
      (dout, srcv, dstv, rows, gsem, ssem, zbuf, obuf, acc,
       ones, dbuf, dacc, dsem) = rest
    else:
      srcv, dstv, rows, gsem, ssem, zbuf, obuf, acc = rest
    cid = lax.axis_index("c")
    sid = lax.axis_index("s")
    wid = cid * NS + sid
    rbase = sid * RPT

    pltpu.sync_copy(srcs.at[wid], srcv)
    pltpu.sync_copy(dsts.at[wid], dstv)

    def zrow(i, _):
      for j in range(D // LANES):
        zbuf[i, pl.ds(j * LANES, LANES)] = jnp.zeros((LANES,), jnp.float32)
      return 0
    lax.fori_loop(0, ZR, zrow, 0)
    if with_deg:
      def zdrow(i, _):
        dbuf[i, :] = jnp.zeros((DEGW,), jnp.float32)
        return 0
      lax.fori_loop(0, RPT // 4, zdrow, 0)
      for i in range(4):
        pltpu.sync_copy(dbuf, dacc.at[pl.ds(rbase + i * (RPT // 4), RPT // 4)])
      def orow(i, _):
        ones[i, :] = jnp.ones((DEGW,), jnp.float32)
        return 0
      lax.fori_loop(0, K, orow, 0)

    for t in range(num_tables):
      if t == 0:
        for i in range(RPT // ZR):
          pltpu.sync_copy(zbuf, acc.at[pl.ds(rbase + i * ZR, ZR)])
        plsc.subcore_barrier()

      first = with_deg and t == 0
      table = tables[t]

      def fire_g(b, c):
        pltpu.async_copy(table.at[srcv.at[c]], rows[b], gsem[b])

      def wait_g(b):
        pltpu.make_async_copy(table.at[srcv.at[0]], rows[b], gsem[b]).wait()

      def fire_s(b, c):
        pltpu.async_copy(rows[b], acc.at[dstv.at[c]], ssem[b], add=True)

      def wait_s(b):
        pltpu.make_async_copy(rows[b], acc.at[dstv.at[0]], ssem[b]).wait()

      def fire_d(c):
        pltpu.async_copy(ones, dacc.at[dstv.at[c]], dsem, add=True)

      for b in range(NB):
        fire_g(b, b)
      def rnd(r, _):
        cb = r * NB
        for b in range(NB):
          wait_g(b)
          fire_s(b, cb + b)
          if first:
            fire_d(cb + b)
        for b in range(NB):
          wait_s(b)
          fire_g(b, cb + NB + b)
        return 0
      lax.fori_loop(0, ROUNDS - 1, rnd, 0)
      cb = (ROUNDS - 1) * NB
      for b in range(NB):
        wait_g(b)
        fire_s(b, cb + b)
        if first:
          fire_d(cb + b)
      for b in range(NB):
        wait_s(b)
      if first:
        def ddrain(c, _):
          pltpu.make_async_copy(ones, dacc.at[dstv.at[0]], dsem).wait()
          return 0
        lax.fori_loop(0, CHUNKS, ddrain, 0)
      plsc.subcore_barrier()

      for i in range(RPT // ZR):
        pltpu.sync_copy(acc.at[pl.ds(rbase + i * ZR, ZR)], obuf)
        pltpu.sync_copy(obuf, outs[t].at[cid, pl.ds(rbase + i * ZR, ZR)])
        if t + 1 < num_tables:
          pltpu.sync_copy(zbuf, acc.at[pl.ds(rbase + i * ZR, ZR)])
      if t + 1 < num_tables:
        plsc.subcore_barrier()

    if with_deg:
      for i in range(4):
        pltpu.sync_copy(dacc.at[pl.ds(rbase + i * (RPT // 4), RPT // 4)], dbuf)
        pltpu.sync_copy(
            dbuf, dout.at[cid, pl.ds(rbase + i * (RPT // 4), RPT // 4)])

  out_type = tuple(out_types) if len(out_types) > 1 else out_types[0]
  return pl.kernel(
      body, out_type=out_type, mesh=mesh, scratch_types=scratch,
      compiler_params=pltpu.CompilerParams(use_tc_tiling_on_sc=False))


_RB = 400


def _mmt(a, w):
  return lax.dot_general(a, w, (((1,), (1,)), ((), ())),
                         preferred_element_type=jnp.float32)


def _tc_mid(aa0, aa1, ab0, ab1, da, db, x, W1l, W1r, b1l, W2l, W2r, b2l):
  def body(aa0_ref, aa1_ref, ab0_ref, ab1_ref, da_ref, db_ref, x_ref,
           w1l_ref, w1r_ref, b1_ref, wl_ref, wr_ref, b2_ref,
           hl_ref, hr_ref):
    deg_b = jnp.maximum((da_ref[0] + db_ref[0])[:, 0:1], 1.0)
    ma = (aa0_ref[0] + aa1_ref[0]) / deg_b
    mb = (ab0_ref[0] + ab1_ref[0]) / deg_b
    w1l = w1l_ref[...]
    lin = (_mmt(ma, w1l[:, :64]) + _mmt(mb, w1l[:, 64:])
           + _mmt(x_ref[...], w1r_ref[...]) + b1_ref[...][None, :])
    h1 = jnp.maximum(lin, 0.0)
    hl_ref[...] = _mmt(h1, wl_ref[...])
    hr_ref[...] = _mmt(h1, wr_ref[...]) + b2_ref[...][None, :]
  s64 = pl.BlockSpec((_RB, 64), lambda i: (i, 0))
  sl0 = pl.BlockSpec((1, _RB, 64), lambda i: (0, i, 0))
  sl1 = pl.BlockSpec((1, _RB, 64), lambda i: (1, i, 0))
  return pl.pallas_call(
      body,
      grid=(N // _RB,),
      in_specs=[
          sl0, sl1, sl0, sl1,
          pl.BlockSpec((1, _RB, DEGW), lambda i: (0, i, 0)),
          pl.BlockSpec((1, _RB, DEGW), lambda i: (1, i, 0)),
          pl.BlockSpec((_RB, 128), lambda i: (i, 0)),
          pl.BlockSpec((128, 128), lambda i: (0, 0)),
          pl.BlockSpec((128, 128), lambda i: (0, 0)),
          pl.BlockSpec((128,), lambda i: (0,)),
          pl.BlockSpec((64, 128), lambda i: (0, 0)),
          pl.BlockSpec((64, 128), lambda i: (0, 0)),
          pl.BlockSpec((64,), lambda i: (0,)),
      ],
      out_specs=[s64, s64],
      out_shape=[jax.ShapeDtypeStruct((N, 64), jnp.float32)] * 2,
  )(aa0, aa1, ab0, ab1, da, db, x, W1l, W1r, b1l, W2l, W2r, b2l)


def _tc_post(aa, ab, da, db, hr, Wout, bout):
  C = Wout.shape[0]
  def body(aa_ref, ab_ref, da_ref, db_ref, hr_ref, wo_ref, bo_ref, out_ref):
    deg_b = jnp.maximum((da_ref[0] + db_ref[0])[:, 0:1], 1.0)
    h2 = jnp.maximum((aa_ref[0] + ab_ref[0]) / deg_b + hr_ref[...], 0.0)
    logits = _mmt(h2, wo_ref[...]) + bo_ref[...][None, :]
    m = jnp.max(logits, axis=1, keepdims=True)
    e = jnp.exp(logits - m)
    out_ref[...] = e / jnp.sum(e, axis=1, keepdims=True)
  s64 = pl.BlockSpec((_RB, 64), lambda i: (i, 0))
  sl0 = pl.BlockSpec((1, _RB, 64), lambda i: (0, i, 0))
  sl1 = pl.BlockSpec((1, _RB, 64), lambda i: (1, i, 0))
  return pl.pallas_call(
      body,
      grid=(N // _RB,),
      in_specs=[
          sl0, sl1,
          pl.BlockSpec((1, _RB, DEGW), lambda i: (0, i, 0)),
          pl.BlockSpec((1, _RB, DEGW), lambda i: (1, i, 0)),
          s64,
          pl.BlockSpec((C, 64), lambda i: (0, 0)),
          pl.BlockSpec((C,), lambda i: (0,)),
      ],
      out_specs=pl.BlockSpec((_RB, C), lambda i: (i, 0)),
      out_shape=jax.ShapeDtypeStruct((N, C), jnp.float32),
  )(aa, ab, da, db, hr, Wout, bout)


@jax.jit
def kernel(x, edge_index, W1l, b1l, W1r, W2l, b2l, W2r, Wout, bout):
  src = edge_index[0].reshape(NW, CHUNKS, K)
  dst = edge_index[1].reshape(NW, CHUNKS, K)

  agg1a, agg1b, deg = _sc_segsum(2, True)(x[:, :64], x[:, 64:], src, dst)
  hl2, hr2 = _tc_mid(agg1a, agg1a, agg1b, agg1b, deg, deg,
                     x, W1l, W1r, b1l, W2l, W2r, b2l)

  agg2 = _sc_segsum(1, False)(hl2, src, dst)
  return _tc_post(agg2, agg2, deg, deg, hr2, Wout, bout)


# speedup vs baseline: 1.0021x; 1.0021x over previous
import jax
import jax.numpy as jnp
from jax import lax
from jax.experimental import pallas as pl
from jax.experimental.pallas import tpu as pltpu
from jax.experimental.pallas import tpu_sc as plsc

N = 10000
N_PAD = 10240
E = 320000
D = 64
NC = 2
NS = 16
LANES = 16
NW = NC * NS
K = 125
CHUNKS = E // (NW * K)
NB = 5
ROUNDS = CHUNKS // NB
RPT = N_PAD // NS
ZR = 64
DEGW = 16


def _sc_segsum(num_tables, with_deg):
  mesh = plsc.VectorSubcoreMesh(
      core_axis_name="c", subcore_axis_name="s",
      num_cores=NC, num_subcores=NS)
  out_types = [jax.ShapeDtypeStruct((NC, N_PAD, D), jnp.float32)
               for _ in range(num_tables)]
  scratch = [
      pltpu.VMEM((CHUNKS, K), jnp.int32),
      pltpu.VMEM((CHUNKS, K), jnp.int32),
      [pltpu.VMEM((K, D), jnp.float32) for _ in range(NB)],
      [pltpu.SemaphoreType.DMA for _ in range(NB)],
      [pltpu.SemaphoreType.DMA for _ in range(NB)],
      pltpu.VMEM((ZR, D), jnp.float32),
      pltpu.VMEM((ZR, D), jnp.float32),
      pltpu.VMEM_SHARED((N_PAD, D), jnp.float32),
  ]
  if with_deg:
    out_types.append(jax.ShapeDtypeStruct((NC, N_PAD, DEGW), jnp.float32))
    scratch += [
        pltpu.VMEM((K, DEGW), jnp.float32),
        pltpu.VMEM((RPT // 4, DEGW), jnp.float32),
        pltpu.VMEM_SHARED((N_PAD, DEGW), jnp.float32),
        pltpu.SemaphoreType.DMA,
    ]

  def body(*refs):
    tables = refs[:num_tables]
    srcs, dsts = refs[num_tables:num_tables + 2]
    rest = refs[num_tables + 2:]
    outs = rest[:num_tables]
    rest = rest[num_tables:]
    if with_deg:
      (dout, srcv, dstv, rows, gsem, ssem, zbuf, obuf, acc,
       ones, dbuf, dacc, dsem) = rest
    else:
      srcv, dstv, rows, gsem, ssem, zbuf, obuf, acc = rest
    cid = lax.axis_index("c")
    sid = lax.axis_index("s")
    wid = cid * NS + sid
    rbase = sid * RPT

    pltpu.sync_copy(srcs.at[wid], srcv)
    pltpu.sync_copy(dsts.at[wid], dstv)

    def zrow(i, _):
      for j in range(D // LANES):
        zbuf[i, pl.ds(j * LANES, LANES)] = jnp.zeros((LANES,), jnp.float32)
      return 0
    lax.fori_loop(0, ZR, zrow, 0)
    if with_deg:
      def zdrow(i, _):
        dbuf[i, :] = jnp.zeros((DEGW,), jnp.float32)
        return 0
      lax.fori_loop(0, RPT // 4, zdrow, 0)
      for i in range(4):
        pltpu.sync_copy(dbuf, dacc.at[pl.ds(rbase + i * (RPT // 4), RPT // 4)])
      def orow(i, _):
        ones[i, :] = jnp.ones((DEGW,), jnp.float32)
        return 0
      lax.fori_loop(0, K, orow, 0)

    for t in range(num_tables):
      for i in range(RPT // ZR):
        pltpu.sync_copy(zbuf, acc.at[pl.ds(rbase + i * ZR, ZR)])
      plsc.subcore_barrier()

      first = with_deg and t == 0
      table = tables[t]

      def fire_g(b, c):
        pltpu.async_copy(table.at[srcv.at[c]], rows[b], gsem[b])

      def wait_g(b):
        pltpu.make_async_copy(table.at[srcv.at[0]], rows[b], gsem[b]).wait()

      def fire_s(b, c):
        pltpu.async_copy(rows[b], acc.at[dstv.at[c]], ssem[b], add=True)

      def wait_s(b):
        pltpu.make_async_copy(rows[b], acc.at[dstv.at[0]], ssem[b]).wait()

      def fire_d(c):
        pltpu.async_copy(ones, dacc.at[dstv.at[c]], dsem, add=True)

      for b in range(NB):
        fire_g(b, b)
      def rnd(r, _):
        cb = r * NB
        for b in range(NB):
          wait_g(b)
          fire_s(b, cb + b)
          if first:
            fire_d(cb + b)
        for b in range(NB):
          wait_s(b)
          fire_g(b, cb + NB + b)
        return 0
      lax.fori_loop(0, ROUNDS - 1, rnd, 0)
      cb = (ROUNDS - 1) * NB
      for b in range(NB):
        wait_g(b)
        fire_s(b, cb + b)
        if first:
          fire_d(cb + b)
      for b in range(NB):
        wait_s(b)
      if first:
        def ddrain(c, _):
          pltpu.make_async_copy(ones, dacc.at[dstv.at[0]], dsem).wait()
          return 0
        lax.fori_loop(0, CHUNKS, ddrain, 0)
      plsc.subcore_barrier()

      for i in range(RPT // ZR):
        pltpu.sync_copy(acc.at[pl.ds(rbase + i * ZR, ZR)], obuf)
        pltpu.sync_copy(obuf, outs[t].at[cid, pl.ds(rbase + i * ZR, ZR)])
      plsc.subcore_barrier()

    if with_deg:
      for i in range(4):
        pltpu.sync_copy(dacc.at[pl.ds(rbase + i * (RPT // 4), RPT // 4)], dbuf)
        pltpu.sync_copy(
            dbuf, dout.at[cid, pl.ds(rbase + i * (RPT // 4), RPT // 4)])

  out_type = tuple(out_types) if len(out_types) > 1 else out_types[0]
  return pl.kernel(
      body, out_type=out_type, mesh=mesh, scratch_types=scratch,
      compiler_params=pltpu.CompilerParams(use_tc_tiling_on_sc=False))


_RB = 400


def _mmt(a, w):
  return lax.dot_general(a, w, (((1,), (1,)), ((), ())),
                         preferred_element_type=jnp.float32)


def _tc_mid(aa0, aa1, ab0, ab1, da, db, x, W1l, W1r, b1l, W2l, W2r, b2l):
  def body(aa0_ref, aa1_ref, ab0_ref, ab1_ref, da_ref, db_ref, x_ref,
           w1l_ref, w1r_ref, b1_ref, wl_ref, wr_ref, b2_ref,
           hl_ref, hr_ref):
    deg_b = jnp.maximum((da_ref[0] + db_ref[0])[:, 0:1], 1.0)
    ma = (aa0_ref[0] + aa1_ref[0]) / deg_b
    mb = (ab0_ref[0] + ab1_ref[0]) / deg_b
    w1l = w1l_ref[...]
    lin = (_mmt(ma, w1l[:, :64]) + _mmt(mb, w1l[:, 64:])
           + _mmt(x_ref[...], w1r_ref[...]) + b1_ref[...][None, :])
    h1 = jnp.maximum(lin, 0.0)
    hl_ref[...] = _mmt(h1, wl_ref[...])
    hr_ref[...] = _mmt(h1, wr_ref[...]) + b2_ref[...][None, :]
  s64 = pl.BlockSpec((_RB, 64), lambda i: (i, 0))
  sl0 = pl.BlockSpec((1, _RB, 64), lambda i: (0, i, 0))
  sl1 = pl.BlockSpec((1, _RB, 64), lambda i: (1, i, 0))
  return pl.pallas_call(
      body,
      grid=(N // _RB,),
      in_specs=[
          sl0, sl1, sl0, sl1,
          pl.BlockSpec((1, _RB, DEGW), lambda i: (0, i, 0)),
          pl.BlockSpec((1, _RB, DEGW), lambda i: (1, i, 0)),
          pl.BlockSpec((_RB, 128), lambda i: (i, 0)),
          pl.BlockSpec((128, 128), lambda i: (0, 0)),
          pl.BlockSpec((128, 128), lambda i: (0, 0)),
          pl.BlockSpec((128,), lambda i: (0,)),
          pl.BlockSpec((64, 128), lambda i: (0, 0)),
          pl.BlockSpec((64, 128), lambda i: (0, 0)),
          pl.BlockSpec((64,), lambda i: (0,)),
      ],
      out_specs=[s64, s64],
      out_shape=[jax.ShapeDtypeStruct((N, 64), jnp.float32)] * 2,
  )(aa0, aa1, ab0, ab1, da, db, x, W1l, W1r, b1l, W2l, W2r, b2l)


def _tc_post(aa, ab, da, db, hr, Wout, bout):
  C = Wout.shape[0]
  def body(aa_ref, ab_ref, da_ref, db_ref, hr_ref, wo_ref, bo_ref, out_ref):
    deg_b = jnp.maximum((da_ref[0] + db_ref[0])[:, 0:1], 1.0)
    h2 = jnp.maximum((aa_ref[0] + ab_ref[0]) / deg_b + hr_ref[...], 0.0)
    logits = _mmt(h2, wo_ref[...]) + bo_ref[...][None, :]
    m = jnp.max(logits, axis=1, keepdims=True)
    e = jnp.exp(logits - m)
    out_ref[...] = e / jnp.sum(e, axis=1, keepdims=True)
  s64 = pl.BlockSpec((_RB, 64), lambda i: (i, 0))
  sl0 = pl.BlockSpec((1, _RB, 64), lambda i: (0, i, 0))
  sl1 = pl.BlockSpec((1, _RB, 64), lambda i: (1, i, 0))
  return pl.pallas_call(
      body,
      grid=(N // _RB,),
      in_specs=[
          sl0, sl1,
          pl.BlockSpec((1, _RB, DEGW), lambda i: (0, i, 0)),
          pl.BlockSpec((1, _RB, DEGW), lambda i: (1, i, 0)),
          s64,
          pl.BlockSpec((C, 64), lambda i: (0, 0)),
          pl.BlockSpec((C,), lambda i: (0,)),
      ],
      out_specs=pl.BlockSpec((_RB, C), lambda i: (i, 0)),
      out_shape=jax.ShapeDtypeStruct((N, C), jnp.float32),
  )(aa, ab, da, db, hr, Wout, bout)


@jax.jit
def kernel(x, edge_index, W1l, b1l, W1r, W2l, b2l, W2r, Wout, bout):
  src = edge_index[0].reshape(NW, CHUNKS, K)
  dst = edge_index[1].reshape(NW, CHUNKS, K)

  agg1a, agg1b, deg = _sc_segsum(2, True)(x[:, :64], x[:, 64:], src, dst)
  hl2, hr2 = _tc_mid(agg1a, agg1a, agg1b, agg1b, deg, deg,
                     x, W1l, W1r, b1l, W2l, W2r, b2l)

  agg2 = _sc_segsum(1, False)(hl2, src, dst)
  return _tc_post(agg2, agg2, deg, deg, hr2, Wout, bout)
